# Initial kernel scaffold; baseline (speedup 1.0000x reference)
import jax
import jax.numpy as jnp
from jax.experimental import pallas as pl


def kernel(bond_dir, bond_type, is_in_ring, W_bond_dir, W_bond_type, W_is_in_ring):
    raise NotImplementedError("write your pallas kernel here")


# speedup vs baseline: 19.0220x; 19.0220x over previous
import functools

import jax
import jax.numpy as jnp
from jax import lax
from jax.experimental import pallas as pl
from jax.experimental.pallas import tpu as pltpu
from jax.experimental.pallas import tpu_sc as plsc

E = 3_200_000
D = 16
V_DIR, V_TYPE, V_RING = 12, 27, 7
NV = V_DIR + V_TYPE + V_RING
NT = V_DIR * V_TYPE * V_RING
NC, NS = 2, 16
NW = NC * NS
EPW = E // NW
CHUNK = 2000
NCHUNK = EPW // CHUNK
GROUPS = CHUNK // 16


@functools.partial(
    pl.kernel,
    mesh=plsc.VectorSubcoreMesh(core_axis_name="c", subcore_axis_name="s"),
    out_type=jax.ShapeDtypeStruct((E * D,), jnp.float32),
    scratch_types=[
        pltpu.VMEM((NV * D,), jnp.float32),
        pltpu.VMEM((NT * D,), jnp.float32),
        pltpu.VMEM((CHUNK,), jnp.int32),
        pltpu.VMEM((CHUNK,), jnp.int32),
        pltpu.VMEM((CHUNK,), jnp.int32),
        pltpu.VMEM((CHUNK * D,), jnp.float32),
        pltpu.SemaphoreType.DMA,
    ],
)
def _sc_lookup(dir_hbm, type_hbm, ring_hbm, w_hbm, out_hbm,
               wv, tv, dirb, typeb, ringb, rows, sem):
    wid = lax.axis_index("s") * NC + lax.axis_index("c")
    tbase = wid * EPW

    pltpu.sync_copy(w_hbm, wv)

    def build_body(r, _):
        i = r // (V_TYPE * V_RING)
        rem = r - i * (V_TYPE * V_RING)
        j = rem // V_RING
        k = rem - j * V_RING
        tv[pl.ds(r * D, D)] = ((wv[pl.ds(i * D, D)]
                                + wv[pl.ds((V_DIR + j) * D, D)])
                               + wv[pl.ds((V_DIR + V_TYPE + k) * D, D)])
        return 0

    lax.fori_loop(0, NT, build_body, 0)

    def chunk_body(ci, _):
        base = pl.multiple_of(tbase + ci * CHUNK, 8)
        pltpu.sync_copy(dir_hbm.at[pl.ds(base, CHUNK)], dirb)
        pltpu.sync_copy(type_hbm.at[pl.ds(base, CHUNK)], typeb)
        pltpu.sync_copy(ring_hbm.at[pl.ds(base, CHUNK)], ringb)

        def group_body(g, _):
            e0 = g * 16
            cv = (dirb[pl.ds(e0, 16)] * (V_TYPE * V_RING)
                  + typeb[pl.ds(e0, 16)] * V_RING
                  + ringb[pl.ds(e0, 16)]) * D
            for u in range(16):
                rows[pl.ds((e0 + u) * D, D)] = tv[pl.ds(cv[u], D)]
            return 0

        lax.fori_loop(0, GROUPS, group_body, 0)

        pltpu.sync_copy(rows, out_hbm.at[pl.ds(base * D, CHUNK * D)])
        return 0

    lax.fori_loop(0, NCHUNK, chunk_body, 0)


def kernel(bond_dir, bond_type, is_in_ring, W_bond_dir, W_bond_type, W_is_in_ring):
    wflat = jnp.concatenate([W_bond_dir.reshape(-1),
                             W_bond_type.reshape(-1),
                             W_is_in_ring.reshape(-1)])
    flat = _sc_lookup(bond_dir, bond_type, is_in_ring, wflat)
    return flat.reshape(E, D)
